# baseline (speedup 1.0000x reference)
import functools

import jax
import jax.numpy as jnp
from jax import lax
from jax.experimental import pallas as pl
from jax.experimental.pallas import tpu as pltpu
from jax.experimental.pallas import tpu_sc as plsc

NNB = 10
NC, NS = 2, 16
LANES = 16
NW = NC * NS
NB = 16
CH = 1568
SROWS = 20
DEPTH = 3


def _feats_sc(msgs_t, counts):
    L, N, D = msgs_t.shape
    NBLK_FULL = CH // NB
    NBLK_LAST = (N - (NW - 1) * CH) // NB
    OUTER = (NBLK_FULL + DEPTH - 1) // DEPTH

    mesh = plsc.VectorSubcoreMesh(
        core_axis_name="c", subcore_axis_name="s",
        num_cores=NC, num_subcores=NS)

    @functools.partial(
        pl.kernel,
        out_type=jax.ShapeDtypeStruct((NNB, N, D), jnp.float32),
        mesh=mesh,
        compiler_params=pltpu.CompilerParams(
            needs_layout_passes=False, use_tc_tiling_on_sc=True),
        scratch_types=[
            pltpu.VMEM((CH + LANES,), jnp.int32),
            pltpu.VMEM((SROWS, NB, D), jnp.float32),
            pltpu.VMEM((SROWS, NB, D), jnp.float32),
            pltpu.VMEM((SROWS, NB, D), jnp.float32),
            pltpu.SemaphoreType.DMA,
            pltpu.SemaphoreType.DMA,
            pltpu.SemaphoreType.DMA,
            pltpu.SemaphoreType.DMA,
            pltpu.SemaphoreType.DMA,
            pltpu.SemaphoreType.DMA,
        ],
    )
    def sc_kernel(msgs_hbm, counts_hbm, feats_out,
                  counts_v, stage0, stage1, stage2,
                  sem_in0, sem_in1, sem_in2, sem_out0, sem_out1, sem_out2):
        wid = lax.axis_index("s") * NC + lax.axis_index("c")
        chunk_base = wid * CH
        nblk = jnp.where(wid == NW - 1, NBLK_LAST, NBLK_FULL)

        base_c = jnp.minimum(chunk_base, N - CH)
        delta = chunk_base - base_c
        pltpu.sync_copy(counts_hbm.at[pl.ds(base_c, CH)],
                        counts_v.at[pl.ds(0, CH)])

        stages = (stage0, stage1, stage2)
        sems_in = (sem_in0, sem_in1, sem_in2)
        sems_out = (sem_out0, sem_out1, sem_out2)
        lane = lax.iota(jnp.int32, LANES)
        zero16 = jnp.zeros((LANES,), jnp.float32)

        def in_drain(slot):
            pltpu.make_async_copy(
                msgs_hbm.at[pl.ds(0, NNB), pl.ds(0, NB), :],
                stages[slot].at[pl.ds(0, NNB), :, :],
                sems_in[slot]).wait()

        def out_copy(slot, bi):
            node0 = chunk_base + bi * NB
            return pltpu.make_async_copy(
                stages[slot].at[pl.ds(0, NNB), :, :],
                feats_out.at[:, pl.ds(node0, NB), :], sems_out[slot])

        def phase(slot, bi):
            prev = (slot + DEPTH - 1) % DEPTH

            @pl.when(bi < nblk)
            def _():
                node0 = chunk_base + bi * NB

                @pl.when(bi >= DEPTH)
                def _():
                    out_copy(slot, bi - DEPTH).wait()

                c16 = plsc.load_gather(
                    counts_v, [delta + bi * NB + lane])
                for t in range(NB):
                    cnt = c16[t]
                    s = jnp.maximum(cnt - NNB, 0)
                    z = jnp.maximum(NNB - cnt, 0)
                    pltpu.async_copy(
                        msgs_hbm.at[pl.ds(s, NNB), pl.ds(node0 + t, 1), :],
                        stages[slot].at[pl.ds(z, NNB), pl.ds(t, 1), :],
                        sems_in[slot])

                @pl.when(bi >= 1)
                def _():
                    in_drain(prev)
                    out_copy(prev, bi - 1).start()

                for t in range(NB):
                    z = jnp.maximum(NNB - c16[t], 0)

                    def zrow(j, carry, t=t):
                        for v in range(D // LANES):
                            stages[slot][j, t,
                                         pl.ds(v * LANES, LANES)] = zero16
                        return carry

                    lax.fori_loop(0, z, zrow, jnp.int32(0))

        def outer(i, carry):
            for k in range(DEPTH):
                phase(k, DEPTH * i + k)
            return carry

        lax.fori_loop(0, OUTER, outer, jnp.int32(0))

        for r in range(DEPTH):
            @pl.when(nblk % DEPTH == r)
            def _(r=r):
                last_slot = (r + DEPTH - 1) % DEPTH
                in_drain(last_slot)
                out_copy(last_slot, nblk - 1).start()

        for k in range(DEPTH):
            out_copy(k, k).wait()

    return sc_kernel(msgs_t, counts)


def _ts_tc(ts_t, counts):
    L, N = ts_t.shape
    B = 2048
    G = pl.cdiv(N, B)
    counts3 = counts.reshape(1, 1, N)

    def tc_kernel(ts_ref, c_ref, out_ref):
        c = c_ref[0, 0, :]
        idx = [c - NNB + j for j in range(NNB)]
        acc = [jnp.full((B,), -1, jnp.int32) for _ in range(NNB)]
        for l in range(L):
            tl = ts_ref[l, :]
            for j in range(NNB):
                acc[j] = jnp.where(idx[j] == l, tl, acc[j])
        for j in range(NNB):
            out_ref[j, :] = acc[j]

    return pl.pallas_call(
        tc_kernel,
        grid=(G,),
        in_specs=[
            pl.BlockSpec((L, B), lambda i: (0, i)),
            pl.BlockSpec((1, 1, B), lambda i: (0, 0, i)),
        ],
        out_specs=pl.BlockSpec((NNB, B), lambda i: (0, i)),
        out_shape=jax.ShapeDtypeStruct((NNB, N), jnp.int32),
    )(ts_t, counts3)


def kernel(msgs, ts, counts):
    feats_t = _feats_sc(jnp.transpose(msgs, (1, 0, 2)), counts)
    ts_o_t = _ts_tc(ts.T, counts)
    return jnp.transpose(feats_t, (1, 0, 2)), ts_o_t.T


# speedup vs baseline: 1.0080x; 1.0080x over previous
import functools

import jax
import jax.numpy as jnp
from jax import lax
from jax.experimental import pallas as pl
from jax.experimental.pallas import tpu as pltpu
from jax.experimental.pallas import tpu_sc as plsc

NNB = 10
NC, NS = 2, 16
LANES = 16
NW = NC * NS
NB = 16
CH = 1568
SROWS = 20


def _feats_sc(msgs_t, counts):
    L, N, D = msgs_t.shape
    NBLK_FULL = CH // NB
    NBLK_LAST = (N - (NW - 1) * CH) // NB
    OUTER = NBLK_FULL // 2

    mesh = plsc.VectorSubcoreMesh(
        core_axis_name="c", subcore_axis_name="s",
        num_cores=NC, num_subcores=NS)

    @functools.partial(
        pl.kernel,
        out_type=jax.ShapeDtypeStruct((NNB, N, D), jnp.float32),
        mesh=mesh,
        compiler_params=pltpu.CompilerParams(
            needs_layout_passes=False, use_tc_tiling_on_sc=True),
        scratch_types=[
            pltpu.VMEM((CH + LANES,), jnp.int32),
            pltpu.VMEM((SROWS, NB, D), jnp.float32),
            pltpu.VMEM((SROWS, NB, D), jnp.float32),
            pltpu.SemaphoreType.DMA,
            pltpu.SemaphoreType.DMA,
            pltpu.SemaphoreType.DMA,
            pltpu.SemaphoreType.DMA,
        ],
    )
    def sc_kernel(msgs_hbm, counts_hbm, feats_out,
                  counts_v, stage0, stage1,
                  sem_in0, sem_in1, sem_out0, sem_out1):
        wid = lax.axis_index("s") * NC + lax.axis_index("c")
        chunk_base = wid * CH
        nblk = jnp.where(wid == NW - 1, NBLK_LAST, NBLK_FULL)

        base_c = jnp.minimum(chunk_base, N - CH)
        delta = chunk_base - base_c
        pltpu.sync_copy(counts_hbm.at[pl.ds(base_c, CH)],
                        counts_v.at[pl.ds(0, CH)])

        stages = (stage0, stage1)
        sems_in = (sem_in0, sem_in1)
        sems_out = (sem_out0, sem_out1)
        lane = lax.iota(jnp.int32, LANES)
        zero16 = jnp.zeros((LANES,), jnp.float32)

        def in_drain(slot):
            pltpu.make_async_copy(
                msgs_hbm.at[pl.ds(0, NNB), pl.ds(0, NB), :],
                stages[slot].at[pl.ds(0, NNB), :, :],
                sems_in[slot]).wait()

        def out_copy(slot, bi):
            node0 = chunk_base + bi * NB
            return pltpu.make_async_copy(
                stages[slot].at[pl.ds(0, NNB), :, :],
                feats_out.at[:, pl.ds(node0, NB), :], sems_out[slot])

        def phase(slot, bi):
            other = 1 - slot

            @pl.when(bi < nblk)
            def _():
                node0 = chunk_base + bi * NB

                @pl.when(bi >= 2)
                def _():
                    out_copy(slot, bi - 2).wait()

                c16 = plsc.load_gather(
                    counts_v, [delta + bi * NB + lane])
                for t in range(NB):
                    cnt = c16[t]
                    s = jnp.maximum(cnt - NNB, 0)
                    z = jnp.maximum(NNB - cnt, 0)
                    pltpu.async_copy(
                        msgs_hbm.at[pl.ds(s, NNB), pl.ds(node0 + t, 1), :],
                        stages[slot].at[pl.ds(z, NNB), pl.ds(t, 1), :],
                        sems_in[slot])
                @pl.when(bi >= 1)
                def _():
                    in_drain(other)
                    out_copy(other, bi - 1).start()

                for t in range(NB):
                    z = jnp.maximum(NNB - c16[t], 0)

                    def zrow(j, carry, t=t):
                        for v in range(D // LANES):
                            stages[slot][j, t,
                                         pl.ds(v * LANES, LANES)] = zero16
                        return carry

                    lax.fori_loop(0, z, zrow, jnp.int32(0))

        def outer(i, carry):
            phase(0, 2 * i)
            phase(1, 2 * i + 1)
            return carry

        lax.fori_loop(0, OUTER, outer, jnp.int32(0))

        @pl.when(nblk % 2 == 0)
        def _():
            in_drain(1)
            out_copy(1, nblk - 1).start()

        @pl.when(nblk % 2 == 1)
        def _():
            in_drain(0)
            out_copy(0, nblk - 1).start()

        out_copy(0, 0).wait()
        out_copy(1, 1).wait()

    return sc_kernel(msgs_t, counts)


def _ts_tc(ts_t, counts):
    L, N = ts_t.shape
    B = 2048
    G = pl.cdiv(N, B)
    counts3 = counts.reshape(1, 1, N)

    def tc_kernel(ts_ref, c_ref, out_ref):
        c = c_ref[0, 0, :]
        idx = [c - NNB + j for j in range(NNB)]
        acc = [jnp.full((B,), -1, jnp.int32) for _ in range(NNB)]
        for l in range(L):
            tl = ts_ref[l, :]
            for j in range(NNB):
                acc[j] = jnp.where(idx[j] == l, tl, acc[j])
        for j in range(NNB):
            out_ref[j, :] = acc[j]

    return pl.pallas_call(
        tc_kernel,
        grid=(G,),
        in_specs=[
            pl.BlockSpec((L, B), lambda i: (0, i)),
            pl.BlockSpec((1, 1, B), lambda i: (0, 0, i)),
        ],
        out_specs=pl.BlockSpec((NNB, B), lambda i: (0, i)),
        out_shape=jax.ShapeDtypeStruct((NNB, N), jnp.int32),
    )(ts_t, counts3)


def kernel(msgs, ts, counts):
    feats_t = _feats_sc(jnp.transpose(msgs, (1, 0, 2)), counts)
    ts_o_t = _ts_tc(ts.T, counts)
    return jnp.transpose(feats_t, (1, 0, 2)), ts_o_t.T
